# initial kernel scaffold (unmeasured)
import jax
import jax.numpy as jnp
from jax import lax
from jax.experimental import pallas as pl
from jax.experimental.pallas import tpu as pltpu

N_Y = 4
B, S, H, Dh, Dr = 2, 256, 16, 64, 32
D = 1024
BS = B * S


def kernel(x, Wdkv, Wuk, Wuv, Wq, Wqr, Wkr, Wo):
    d_c = Wdkv.shape[1]

    def body(x_ref, wdkv_ref, wuk_ref, wuv_ref, wq_ref, wqr_ref, wkr_ref,
             wo_ref, out_ref, comm_ref, send_sems, recv_sems):
        my_x = lax.axis_index("x")
        my_y = lax.axis_index("y")
        my_z = lax.axis_index("z")
        left = lax.rem(my_y + (N_Y - 1), N_Y)
        right = lax.rem(my_y + 1, N_Y)

        barrier_sem = pltpu.get_barrier_semaphore()
        for nbr in (left, right):
            pl.semaphore_signal(
                barrier_sem, inc=1,
                device_id=(my_x, nbr, my_z),
                device_id_type=pl.DeviceIdType.MESH,
            )
        pl.semaphore_wait(barrier_sem, 2)

        xf = x_ref[...].reshape(BS, D).astype(jnp.bfloat16)
        c = jnp.dot(xf, wdkv_ref[...].astype(jnp.bfloat16),
                    preferred_element_type=jnp.float32).astype(jnp.bfloat16)
        kp = jnp.dot(c, wuk_ref[...].astype(jnp.bfloat16),
                     preferred_element_type=jnp.float32)
        vp = jnp.dot(c, wuv_ref[...].astype(jnp.bfloat16),
                     preferred_element_type=jnp.float32)
        comm_ref[0, 0] = kp.astype(jnp.bfloat16)
        comm_ref[0, 1] = vp.astype(jnp.bfloat16)

        for h in range(N_Y - 1):
            rdma = pltpu.make_async_remote_copy(
                src_ref=comm_ref.at[h],
                dst_ref=comm_ref.at[h + 1],
                send_sem=send_sems.at[h],
                recv_sem=recv_sems.at[h],
                device_id=(my_x, right, my_z),
                device_id_type=pl.DeviceIdType.MESH,
            )
            rdma.start()
            if h == 0:
                q2d = jnp.dot(xf, wq_ref[...].astype(jnp.bfloat16),
                              preferred_element_type=jnp.float32
                              ).astype(jnp.bfloat16)
                qr2d = jnp.dot(xf, wqr_ref[...].astype(jnp.bfloat16),
                               preferred_element_type=jnp.float32
                               ).astype(jnp.bfloat16)
                kr2d = jnp.dot(xf, wkr_ref[...].astype(jnp.bfloat16),
                               preferred_element_type=jnp.float32
                               ).astype(jnp.bfloat16)
            rdma.wait()

        ksum = (comm_ref[0, 0].astype(jnp.float32)
                + comm_ref[1, 0].astype(jnp.float32)
                + comm_ref[2, 0].astype(jnp.float32)
                + comm_ref[3, 0].astype(jnp.float32))
        vsum = (comm_ref[0, 1].astype(jnp.float32)
                + comm_ref[1, 1].astype(jnp.float32)
                + comm_ref[2, 1].astype(jnp.float32)
                + comm_ref[3, 1].astype(jnp.float32))
        k2d = ksum.astype(jnp.bfloat16)
        v2d = vsum.astype(jnp.bfloat16)

        scale = (Dh + Dr) ** -0.5
        wo_bf = wo_ref[...].astype(jnp.bfloat16)
        for b in range(B):
            rows = pl.ds(b * S, S)
            o_heads = []
            for h in range(H):
                qh = q2d[rows, h * Dh:(h + 1) * Dh]
                kh = k2d[rows, h * Dh:(h + 1) * Dh]
                vh = v2d[rows, h * Dh:(h + 1) * Dh]
                qrh = qr2d[rows, h * Dr:(h + 1) * Dr]
                krh = kr2d[rows, :]
                s1 = lax.dot_general(qh, kh, (((1,), (1,)), ((), ())),
                                     preferred_element_type=jnp.float32)
                s2 = lax.dot_general(qrh, krh, (((1,), (1,)), ((), ())),
                                     preferred_element_type=jnp.float32)
                sc = (s1 + s2) * scale
                m = jnp.max(sc, axis=-1, keepdims=True)
                p = jnp.exp(sc - m)
                p = p / jnp.sum(p, axis=-1, keepdims=True)
                oh = jnp.dot(p.astype(jnp.bfloat16), vh,
                             preferred_element_type=jnp.float32)
                o_heads.append(oh.astype(jnp.bfloat16))
            o_b = jnp.concatenate(o_heads, axis=-1)
            out_ref[b] = jnp.dot(o_b, wo_bf,
                                 preferred_element_type=jnp.float32)

    out_shape = jax.ShapeDtypeStruct((B, S, D), jnp.float32)
    return pl.pallas_call(
        body,
        out_shape=out_shape,
        in_specs=[pl.BlockSpec(memory_space=pltpu.VMEM)] * 8,
        out_specs=pl.BlockSpec(memory_space=pltpu.VMEM),
        scratch_shapes=[
            pltpu.VMEM((N_Y, 2, BS, D), jnp.bfloat16),
            pltpu.SemaphoreType.DMA((N_Y - 1,)),
            pltpu.SemaphoreType.DMA((N_Y - 1,)),
        ],
        compiler_params=pltpu.CompilerParams(collective_id=0),
    )(x, Wdkv, Wuk, Wuv, Wq, Wqr, Wkr, Wo)


# baseline (device time: 106913 ns/iter reference)
import jax
import jax.numpy as jnp
from jax import lax
from jax.experimental import pallas as pl
from jax.experimental.pallas import tpu as pltpu

N_Y = 4
B, S, H, Dh, Dr = 2, 256, 16, 64, 32
D = 1024
BS = B * S


def kernel(x, Wdkv, Wuk, Wuv, Wq, Wqr, Wkr, Wo):
    d_c = Wdkv.shape[1]

    def body(x_ref, wdkv_ref, wuk_ref, wuv_ref, wq_ref, wqr_ref, wkr_ref,
             wo_ref, out_ref, comm_ref, send_sems, recv_sems):
        my_x = lax.axis_index("x")
        my_y = lax.axis_index("y")
        my_z = lax.axis_index("z")
        left = lax.rem(my_y + (N_Y - 1), N_Y)
        right = lax.rem(my_y + 1, N_Y)

        barrier_sem = pltpu.get_barrier_semaphore()
        for nbr in (left, right):
            pl.semaphore_signal(
                barrier_sem, inc=1,
                device_id=(my_x, nbr, my_z),
                device_id_type=pl.DeviceIdType.MESH,
            )
        pl.semaphore_wait(barrier_sem, 2)

        xf = x_ref[...].reshape(BS, D).astype(jnp.bfloat16)
        c = jnp.dot(xf, wdkv_ref[...].astype(jnp.bfloat16),
                    preferred_element_type=jnp.float32).astype(jnp.bfloat16)
        kp = jnp.dot(c, wuk_ref[...].astype(jnp.bfloat16),
                     preferred_element_type=jnp.float32)
        vp = jnp.dot(c, wuv_ref[...].astype(jnp.bfloat16),
                     preferred_element_type=jnp.float32)
        comm_ref[0, 0] = kp.astype(jnp.bfloat16)
        comm_ref[0, 1] = vp.astype(jnp.bfloat16)

        for h in range(N_Y - 1):
            rdma = pltpu.make_async_remote_copy(
                src_ref=comm_ref.at[h],
                dst_ref=comm_ref.at[h + 1],
                send_sem=send_sems.at[h],
                recv_sem=recv_sems.at[h],
                device_id=(my_x, right, my_z),
                device_id_type=pl.DeviceIdType.MESH,
            )
            rdma.start()
            if h == 0:
                q2d = jnp.dot(xf, wq_ref[...].astype(jnp.bfloat16),
                              preferred_element_type=jnp.float32
                              ).astype(jnp.bfloat16)
                qr2d = jnp.dot(xf, wqr_ref[...].astype(jnp.bfloat16),
                               preferred_element_type=jnp.float32
                               ).astype(jnp.bfloat16)
                kr2d = jnp.dot(xf, wkr_ref[...].astype(jnp.bfloat16),
                               preferred_element_type=jnp.float32
                               ).astype(jnp.bfloat16)
            rdma.wait()

        ksum = (comm_ref[0, 0].astype(jnp.float32)
                + comm_ref[1, 0].astype(jnp.float32)
                + comm_ref[2, 0].astype(jnp.float32)
                + comm_ref[3, 0].astype(jnp.float32))
        vsum = (comm_ref[0, 1].astype(jnp.float32)
                + comm_ref[1, 1].astype(jnp.float32)
                + comm_ref[2, 1].astype(jnp.float32)
                + comm_ref[3, 1].astype(jnp.float32))
        k2d = ksum.astype(jnp.bfloat16)
        v2d = vsum.astype(jnp.bfloat16)

        scale = (Dh + Dr) ** -0.5
        wo_bf = wo_ref[...].astype(jnp.bfloat16)
        for b in range(B):
            r0, r1 = b * S, (b + 1) * S
            o_heads = []
            for h in range(H):
                qh = q2d[r0:r1, h * Dh:(h + 1) * Dh]
                kh = k2d[r0:r1, h * Dh:(h + 1) * Dh]
                vh = v2d[r0:r1, h * Dh:(h + 1) * Dh]
                qrh = qr2d[r0:r1, h * Dr:(h + 1) * Dr]
                krh = kr2d[r0:r1, :]
                s1 = lax.dot_general(qh, kh, (((1,), (1,)), ((), ())),
                                     preferred_element_type=jnp.float32)
                s2 = lax.dot_general(qrh, krh, (((1,), (1,)), ((), ())),
                                     preferred_element_type=jnp.float32)
                sc = (s1 + s2) * scale
                m = jnp.max(sc, axis=-1, keepdims=True)
                p = jnp.exp(sc - m)
                p = p / jnp.sum(p, axis=-1, keepdims=True)
                oh = jnp.dot(p.astype(jnp.bfloat16), vh,
                             preferred_element_type=jnp.float32)
                o_heads.append(oh.astype(jnp.bfloat16))
            o_b = jnp.concatenate(o_heads, axis=-1)
            out_ref[b] = jnp.dot(o_b, wo_bf,
                                 preferred_element_type=jnp.float32)

    out_shape = jax.ShapeDtypeStruct((B, S, D), jnp.float32)
    return pl.pallas_call(
        body,
        out_shape=out_shape,
        in_specs=[pl.BlockSpec(memory_space=pltpu.VMEM)] * 8,
        out_specs=pl.BlockSpec(memory_space=pltpu.VMEM),
        scratch_shapes=[
            pltpu.VMEM((N_Y, 2, BS, D), jnp.bfloat16),
            pltpu.SemaphoreType.DMA((N_Y - 1,)),
            pltpu.SemaphoreType.DMA((N_Y - 1,)),
        ],
        compiler_params=pltpu.CompilerParams(collective_id=0),
    )(x, Wdkv, Wuk, Wuv, Wq, Wqr, Wkr, Wo)


# device time: 53765 ns/iter; 1.9885x vs baseline; 1.9885x over previous
import jax
import jax.numpy as jnp
from jax import lax
from jax.experimental import pallas as pl
from jax.experimental.pallas import tpu as pltpu

N_Y = 4
B, S, H, Dh, Dr = 2, 256, 16, 64, 32
D = 1024
BS = B * S


def kernel(x, Wdkv, Wuk, Wuv, Wq, Wqr, Wkr, Wo):
    d_c = Wdkv.shape[1]

    def body(x_ref, wdkv_ref, wuk_ref, wuv_ref, wq_ref, wqr_ref, wkr_ref,
             wo_ref, out_ref, cbuf, kwbuf, vwbuf, send_sems, recv_sems):
        my_x = lax.axis_index("x")
        my_y = lax.axis_index("y")
        my_z = lax.axis_index("z")
        left = lax.rem(my_y + (N_Y - 1), N_Y)
        right = lax.rem(my_y + 1, N_Y)

        barrier_sem = pltpu.get_barrier_semaphore()
        for nbr in (left, right):
            pl.semaphore_signal(
                barrier_sem, inc=1,
                device_id=(my_x, nbr, my_z),
                device_id_type=pl.DeviceIdType.MESH,
            )
        pl.semaphore_wait(barrier_sem, 2)

        xf = x_ref[...].reshape(BS, D).astype(jnp.bfloat16)
        c = jnp.dot(xf, wdkv_ref[...].astype(jnp.bfloat16),
                    preferred_element_type=jnp.float32).astype(jnp.bfloat16)
        cbuf[0] = c
        kwbuf[0] = wuk_ref[...].astype(jnp.bfloat16)
        vwbuf[0] = wuv_ref[...].astype(jnp.bfloat16)

        for h in range(N_Y - 1):
            rdmas = []
            for t, buf in enumerate((cbuf, kwbuf, vwbuf)):
                rdma = pltpu.make_async_remote_copy(
                    src_ref=buf.at[h],
                    dst_ref=buf.at[h + 1],
                    send_sem=send_sems.at[h, t],
                    recv_sem=recv_sems.at[h, t],
                    device_id=(my_x, right, my_z),
                    device_id_type=pl.DeviceIdType.MESH,
                )
                rdma.start()
                rdmas.append(rdma)
            if h == 0:
                q2d = jnp.dot(xf, wq_ref[...].astype(jnp.bfloat16),
                              preferred_element_type=jnp.float32
                              ).astype(jnp.bfloat16)
                qr2d = jnp.dot(xf, wqr_ref[...].astype(jnp.bfloat16),
                               preferred_element_type=jnp.float32
                               ).astype(jnp.bfloat16)
                kr2d = jnp.dot(xf, wkr_ref[...].astype(jnp.bfloat16),
                               preferred_element_type=jnp.float32
                               ).astype(jnp.bfloat16)
            for rdma in rdmas:
                rdma.wait()

        ksum = jnp.dot(cbuf[0], kwbuf[0], preferred_element_type=jnp.float32)
        vsum = jnp.dot(cbuf[0], vwbuf[0], preferred_element_type=jnp.float32)
        for s in range(1, N_Y):
            ksum = ksum + jnp.dot(cbuf[s], kwbuf[s],
                                  preferred_element_type=jnp.float32)
            vsum = vsum + jnp.dot(cbuf[s], vwbuf[s],
                                  preferred_element_type=jnp.float32)
        k2d = ksum.astype(jnp.bfloat16)
        v2d = vsum.astype(jnp.bfloat16)

        scale = (Dh + Dr) ** -0.5
        wo_bf = wo_ref[...].astype(jnp.bfloat16)
        for b in range(B):
            r0, r1 = b * S, (b + 1) * S
            o_heads = []
            for h in range(H):
                qh = q2d[r0:r1, h * Dh:(h + 1) * Dh]
                kh = k2d[r0:r1, h * Dh:(h + 1) * Dh]
                vh = v2d[r0:r1, h * Dh:(h + 1) * Dh]
                qrh = qr2d[r0:r1, h * Dr:(h + 1) * Dr]
                krh = kr2d[r0:r1, :]
                s1 = lax.dot_general(qh, kh, (((1,), (1,)), ((), ())),
                                     preferred_element_type=jnp.float32)
                s2 = lax.dot_general(qrh, krh, (((1,), (1,)), ((), ())),
                                     preferred_element_type=jnp.float32)
                sc = (s1 + s2) * scale
                m = jnp.max(sc, axis=-1, keepdims=True)
                p = jnp.exp(sc - m)
                p = p / jnp.sum(p, axis=-1, keepdims=True)
                oh = jnp.dot(p.astype(jnp.bfloat16), vh,
                             preferred_element_type=jnp.float32)
                o_heads.append(oh.astype(jnp.bfloat16))
            o_b = jnp.concatenate(o_heads, axis=-1)
            out_ref[b] = jnp.dot(o_b, wo_bf,
                                 preferred_element_type=jnp.float32)

    out_shape = jax.ShapeDtypeStruct((B, S, D), jnp.float32)
    return pl.pallas_call(
        body,
        out_shape=out_shape,
        in_specs=[pl.BlockSpec(memory_space=pltpu.VMEM)] * 8,
        out_specs=pl.BlockSpec(memory_space=pltpu.VMEM),
        scratch_shapes=[
            pltpu.VMEM((N_Y, BS, d_c), jnp.bfloat16),
            pltpu.VMEM((N_Y, d_c, D), jnp.bfloat16),
            pltpu.VMEM((N_Y, d_c, D), jnp.bfloat16),
            pltpu.SemaphoreType.DMA((N_Y - 1, 3)),
            pltpu.SemaphoreType.DMA((N_Y - 1, 3)),
        ],
        compiler_params=pltpu.CompilerParams(collective_id=0),
    )(x, Wdkv, Wuk, Wuv, Wq, Wqr, Wkr, Wo)


# device time: 43146 ns/iter; 2.4779x vs baseline; 1.2461x over previous
import os

import jax
import jax.numpy as jnp
from jax import lax
from jax.experimental import pallas as pl
from jax.experimental.pallas import tpu as pltpu

_SKIP_RING = bool(os.environ.get("SKIP_RING"))

N_Y = 4
B, S, H, Dh, Dr = 2, 256, 16, 64, 32
D = 1024
BS = B * S


def kernel(x, Wdkv, Wuk, Wuv, Wq, Wqr, Wkr, Wo):
    d_c = Wdkv.shape[1]

    def body(x_ref, wdkv_ref, wuk_ref, wuv_ref, wq_ref, wqr_ref, wkr_ref,
             wo_ref, out_ref, cbuf, kwbuf, vwbuf, send_sems, recv_sems):
        my_x = lax.axis_index("x")
        my_y = lax.axis_index("y")
        my_z = lax.axis_index("z")

        if not _SKIP_RING:
            barrier_sem = pltpu.get_barrier_semaphore()
            for d in range(1, N_Y):
                pl.semaphore_signal(
                    barrier_sem, inc=1,
                    device_id=(my_x, lax.rem(my_y + d, N_Y), my_z),
                    device_id_type=pl.DeviceIdType.MESH,
                )
            pl.semaphore_wait(barrier_sem, N_Y - 1)

        xf = x_ref[...].reshape(BS, D).astype(jnp.bfloat16)
        c = jnp.dot(xf, wdkv_ref[...].astype(jnp.bfloat16),
                    preferred_element_type=jnp.float32).astype(jnp.bfloat16)
        cbuf[0] = c
        kwbuf[0] = wuk_ref[...].astype(jnp.bfloat16)
        vwbuf[0] = wuv_ref[...].astype(jnp.bfloat16)

        rdmas = []
        if not _SKIP_RING:
            for d in range(1, N_Y):
                for t, buf in enumerate((cbuf, kwbuf, vwbuf)):
                    rdma = pltpu.make_async_remote_copy(
                        src_ref=buf.at[0],
                        dst_ref=buf.at[d],
                        send_sem=send_sems.at[d - 1, t],
                        recv_sem=recv_sems.at[d - 1, t],
                        device_id=(my_x, lax.rem(my_y + d, N_Y), my_z),
                        device_id_type=pl.DeviceIdType.MESH,
                    )
                    rdma.start()
                    rdmas.append(rdma)

        scale = (Dh + Dr) ** -0.5
        q2d = (jnp.dot(xf, wq_ref[...].astype(jnp.bfloat16),
                       preferred_element_type=jnp.float32) * scale
               ).astype(jnp.bfloat16)
        qr2d = (jnp.dot(xf, wqr_ref[...].astype(jnp.bfloat16),
                        preferred_element_type=jnp.float32) * scale
                ).astype(jnp.bfloat16)
        kr2d = jnp.dot(xf, wkr_ref[...].astype(jnp.bfloat16),
                       preferred_element_type=jnp.float32
                       ).astype(jnp.bfloat16)
        s2 = [[lax.dot_general(
                   qr2d[b * S:(b + 1) * S, h * Dr:(h + 1) * Dr],
                   kr2d[b * S:(b + 1) * S, :],
                   (((1,), (1,)), ((), ())),
                   preferred_element_type=jnp.float32)
               for h in range(H)] for b in range(B)]

        for rdma in rdmas:
            rdma.wait()

        c_full = jnp.concatenate([cbuf[s] for s in range(N_Y)], axis=1)
        wuk_full = jnp.concatenate([kwbuf[s] for s in range(N_Y)], axis=0)
        wuv_full = jnp.concatenate([vwbuf[s] for s in range(N_Y)], axis=0)
        k2d = jnp.dot(c_full, wuk_full,
                      preferred_element_type=jnp.float32
                      ).astype(jnp.bfloat16)
        v2d = jnp.dot(c_full, wuv_full,
                      preferred_element_type=jnp.float32
                      ).astype(jnp.bfloat16)

        wo_bf = wo_ref[...].astype(jnp.bfloat16)
        for b in range(B):
            r0, r1 = b * S, (b + 1) * S
            o_heads = []
            for h in range(H):
                qh = q2d[r0:r1, h * Dh:(h + 1) * Dh]
                kh = k2d[r0:r1, h * Dh:(h + 1) * Dh]
                vh = v2d[r0:r1, h * Dh:(h + 1) * Dh]
                sc = lax.dot_general(qh, kh, (((1,), (1,)), ((), ())),
                                     preferred_element_type=jnp.float32
                                     ) + s2[b][h]
                p = jnp.exp(sc)
                z = jnp.sum(p, axis=-1, keepdims=True)
                oh = lax.dot_general(p.astype(jnp.bfloat16), vh,
                                     (((1,), (0,)), ((), ())),
                                     preferred_element_type=jnp.float32)
                oh = oh * (1.0 / z)
                o_heads.append(oh.astype(jnp.bfloat16))
            o_b = jnp.concatenate(o_heads, axis=-1)
            out_ref[b] = jnp.dot(o_b, wo_bf,
                                 preferred_element_type=jnp.float32)

    out_shape = jax.ShapeDtypeStruct((B, S, D), jnp.float32)
    return pl.pallas_call(
        body,
        out_shape=out_shape,
        in_specs=[pl.BlockSpec(memory_space=pltpu.VMEM)] * 8,
        out_specs=pl.BlockSpec(memory_space=pltpu.VMEM),
        scratch_shapes=[
            pltpu.VMEM((N_Y, BS, d_c), jnp.bfloat16),
            pltpu.VMEM((N_Y, d_c, D), jnp.bfloat16),
            pltpu.VMEM((N_Y, d_c, D), jnp.bfloat16),
            pltpu.SemaphoreType.DMA((N_Y - 1, 3)),
            pltpu.SemaphoreType.DMA((N_Y - 1, 3)),
        ],
        compiler_params=(None if _SKIP_RING
                         else pltpu.CompilerParams(collective_id=0)),
    )(x, Wdkv, Wuk, Wuv, Wq, Wqr, Wkr, Wo)


# device time: 38763 ns/iter; 2.7581x vs baseline; 1.1131x over previous
import os

import jax
import jax.numpy as jnp
from jax import lax
from jax.experimental import pallas as pl
from jax.experimental.pallas import tpu as pltpu

_SKIP_RING = bool(os.environ.get("SKIP_RING"))

N_Y = 4
B, S, H, Dh, Dr = 2, 256, 16, 64, 32
D = 1024
BS = B * S


def kernel(x, Wdkv, Wuk, Wuv, Wq, Wqr, Wkr, Wo):
    d_c = Wdkv.shape[1]

    def body(x_ref, wdkv_ref, wuk_ref, wuv_ref, wq_ref, wqr_ref, wkr_ref,
             wo_ref, out_ref, cbuf, kwbuf, vwbuf, send_sems, recv_sems):
        my_x = lax.axis_index("x")
        my_y = lax.axis_index("y")
        my_z = lax.axis_index("z")

        if not _SKIP_RING:
            barrier_sem = pltpu.get_barrier_semaphore()
            for d in range(1, N_Y):
                pl.semaphore_signal(
                    barrier_sem, inc=1,
                    device_id=(my_x, lax.rem(my_y + d, N_Y), my_z),
                    device_id_type=pl.DeviceIdType.MESH,
                )
            pl.semaphore_wait(barrier_sem, N_Y - 1)

        xf = x_ref[...].reshape(BS, D).astype(jnp.bfloat16)
        c = jnp.dot(xf, wdkv_ref[...].astype(jnp.bfloat16),
                    preferred_element_type=jnp.float32).astype(jnp.bfloat16)
        cbuf[0] = c
        kwbuf[0] = wuk_ref[...].astype(jnp.bfloat16)
        vwbuf[0] = wuv_ref[...].astype(jnp.bfloat16)

        def start_peer_sends(t, buf):
            rdmas = []
            for d in range(1, N_Y):
                rdma = pltpu.make_async_remote_copy(
                    src_ref=buf.at[0],
                    dst_ref=buf.at[d],
                    send_sem=send_sems.at[d - 1, t],
                    recv_sem=recv_sems.at[d - 1, t],
                    device_id=(my_x, lax.rem(my_y + d, N_Y), my_z),
                    device_id_type=pl.DeviceIdType.MESH,
                )
                rdma.start()
                rdmas.append(rdma)
            return rdmas

        rdmas_ck = []
        if not _SKIP_RING:
            rdmas_ck = start_peer_sends(0, cbuf) + start_peer_sends(1, kwbuf)

        scale = (Dh + Dr) ** -0.5
        q2d = (jnp.dot(xf, wq_ref[...].astype(jnp.bfloat16),
                       preferred_element_type=jnp.float32) * scale
               ).astype(jnp.bfloat16)
        qr2d = (jnp.dot(xf, wqr_ref[...].astype(jnp.bfloat16),
                        preferred_element_type=jnp.float32) * scale
                ).astype(jnp.bfloat16)
        kr2d = jnp.dot(xf, wkr_ref[...].astype(jnp.bfloat16),
                       preferred_element_type=jnp.float32
                       ).astype(jnp.bfloat16)
        s2 = [[lax.dot_general(
                   qr2d[b * S:(b + 1) * S, h * Dr:(h + 1) * Dr],
                   kr2d[b * S:(b + 1) * S, :],
                   (((1,), (1,)), ((), ())),
                   preferred_element_type=jnp.float32)
               for h in range(H)] for b in range(B)]

        for rdma in rdmas_ck:
            rdma.wait()
        rdmas_v = [] if _SKIP_RING else start_peer_sends(2, vwbuf)

        c_full = jnp.concatenate([cbuf[s] for s in range(N_Y)], axis=1)
        wuk_full = jnp.concatenate([kwbuf[s] for s in range(N_Y)], axis=0)
        k2d = jnp.dot(c_full, wuk_full,
                      preferred_element_type=jnp.float32
                      ).astype(jnp.bfloat16)

        wo_bf = wo_ref[...].astype(jnp.bfloat16)
        p_all = [[None] * H for _ in range(B)]
        zinv_all = [[None] * H for _ in range(B)]
        for b in range(B):
            r0, r1 = b * S, (b + 1) * S
            for h in range(H):
                qh = q2d[r0:r1, h * Dh:(h + 1) * Dh]
                kh = k2d[r0:r1, h * Dh:(h + 1) * Dh]
                sc = lax.dot_general(qh, kh, (((1,), (1,)), ((), ())),
                                     preferred_element_type=jnp.float32
                                     ) + s2[b][h]
                p = jnp.exp(sc)
                z = jnp.sum(p, axis=-1, keepdims=True)
                p_all[b][h] = p.astype(jnp.bfloat16)
                zinv_all[b][h] = 1.0 / z

        for rdma in rdmas_v:
            rdma.wait()
        wuv_full = jnp.concatenate([vwbuf[s] for s in range(N_Y)], axis=0)
        v2d = jnp.dot(c_full, wuv_full,
                      preferred_element_type=jnp.float32
                      ).astype(jnp.bfloat16)

        for b in range(B):
            r0, r1 = b * S, (b + 1) * S
            o_heads = []
            for h in range(H):
                vh = v2d[r0:r1, h * Dh:(h + 1) * Dh]
                oh = lax.dot_general(p_all[b][h], vh,
                                     (((1,), (0,)), ((), ())),
                                     preferred_element_type=jnp.float32)
                oh = oh * zinv_all[b][h]
                o_heads.append(oh.astype(jnp.bfloat16))
            o_b = jnp.concatenate(o_heads, axis=-1)
            out_ref[b] = jnp.dot(o_b, wo_bf,
                                 preferred_element_type=jnp.float32)

    out_shape = jax.ShapeDtypeStruct((B, S, D), jnp.float32)
    return pl.pallas_call(
        body,
        out_shape=out_shape,
        in_specs=[pl.BlockSpec(memory_space=pltpu.VMEM)] * 8,
        out_specs=pl.BlockSpec(memory_space=pltpu.VMEM),
        scratch_shapes=[
            pltpu.VMEM((N_Y, BS, d_c), jnp.bfloat16),
            pltpu.VMEM((N_Y, d_c, D), jnp.bfloat16),
            pltpu.VMEM((N_Y, d_c, D), jnp.bfloat16),
            pltpu.SemaphoreType.DMA((N_Y - 1, 3)),
            pltpu.SemaphoreType.DMA((N_Y - 1, 3)),
        ],
        compiler_params=(None if _SKIP_RING
                         else pltpu.CompilerParams(collective_id=0)),
    )(x, Wdkv, Wuk, Wuv, Wq, Wqr, Wkr, Wo)
